# baseline (speedup 1.0000x reference)
import jax
import jax.numpy as jnp
import numpy as np
from jax.experimental import pallas as pl
from jax.experimental.pallas import tpu as pltpu

_THRESHOLDS = [float(t) for t in np.arange(0.1, 0.91, 0.01).astype(np.float32)]
_HW = 32 * 32


def _conv2(a, w, s):
    return jax.lax.conv_general_dilated(
        a, w, (s, s), 'SAME', dimension_numbers=('NHWC', 'HWIO', 'NHWC'))


def _convT2(a, w, s):
    return jax.lax.conv_transpose(
        a, w, (s, s), 'SAME', dimension_numbers=('NHWC', 'HWIO', 'NHWC'))


def _mid_kernel(h_ref, eps_ref, wzm_ref, bzm_ref, wzl_ref, bzl_ref,
                dw_ref, db_ref, o_ref):
    h = h_ref[...]
    zm = jnp.dot(h, wzm_ref[...], preferred_element_type=jnp.float32)
    zm = jnp.maximum(zm + bzm_ref[...], 0.0)
    zl = jnp.dot(h, wzl_ref[...], preferred_element_type=jnp.float32)
    zl = jnp.maximum(zl + bzl_ref[...], 0.0)
    z = zm + jnp.exp(0.5 * zl) * eps_ref[...]
    d = jnp.dot(z, dw_ref[...], preferred_element_type=jnp.float32)
    o_ref[...] = jnp.maximum(d + db_ref[...], 0.0)


def _otsu_kernel(t_ref, x_ref, p_ref, o_ref):
    recon = jax.nn.sigmoid(p_ref[...])
    diff = x_ref[...] - recon
    err = jnp.mean(diff * diff, axis=1)

    zero = jnp.zeros((err.shape[0], 1), jnp.float32)

    def body(i, carry):
        sig_max, opti = carry
        t = t_ref[i]
        below = err < t
        e0 = jnp.where(below, err, 0.0)
        e1 = jnp.where(below, 0.0, err)
        n0 = jnp.sum((e0 != 0.0).astype(jnp.float32), axis=1, keepdims=True)
        n1 = jnp.sum((e1 != 0.0).astype(jnp.float32), axis=1, keepdims=True)
        s0 = jnp.sum(e0, axis=1, keepdims=True)
        s1 = jnp.sum(e1, axis=1, keepdims=True)
        m0 = (s0 / _HW) * ((n1 + n0) / n0)
        m0 = jnp.where(jnp.isnan(m0), 0.0, m0)
        m1 = (s1 / _HW) * ((n1 + n0) / n1)
        m1 = jnp.where(jnp.isnan(m1), 0.0, m1)
        p0 = n0 / (n0 + n1)
        p1 = n1 / (n0 + n1)
        sig_b = p0 * p1 * (m0 - m1) ** 2
        upd = sig_b >= sig_max
        return (jnp.where(upd, sig_b, sig_max), jnp.where(upd, t, opti))

    _, opti = jax.lax.fori_loop(0, len(_THRESHOLDS), body, (zero, zero))
    o_ref[...] = (err < opti).astype(jnp.int32)


def _dense_middle(h, eps, wzm, bzm, wzl, bzl, dw, db, bblk):
    b = h.shape[0]
    grid = (b // bblk,)
    return pl.pallas_call(
        _mid_kernel,
        grid=grid,
        in_specs=[
            pl.BlockSpec((bblk, 2048), lambda i: (i, 0)),
            pl.BlockSpec((bblk, 128), lambda i: (i, 0)),
            pl.BlockSpec((2048, 128), lambda i: (0, 0)),
            pl.BlockSpec((1, 128), lambda i: (0, 0)),
            pl.BlockSpec((2048, 128), lambda i: (0, 0)),
            pl.BlockSpec((1, 128), lambda i: (0, 0)),
            pl.BlockSpec((128, 8192), lambda i: (0, 0)),
            pl.BlockSpec((1, 8192), lambda i: (0, 0)),
        ],
        out_specs=pl.BlockSpec((bblk, 8192), lambda i: (i, 0)),
        out_shape=jax.ShapeDtypeStruct((b, 8192), jnp.float32),
        compiler_params=pltpu.CompilerParams(
            dimension_semantics=("parallel",)),
    )(h, eps, wzm, bzm.reshape(1, -1), wzl, bzl.reshape(1, -1),
      dw, db.reshape(1, -1))


def _otsu_mask(x, pre, bblk):
    b = x.shape[0]
    xt = x.reshape(b, 1024, 3).transpose(0, 2, 1)
    pt = pre.reshape(b, 1024, 3).transpose(0, 2, 1)
    grid = (b // bblk,)
    thresholds = jnp.asarray(np.arange(0.1, 0.91, 0.01), dtype=jnp.float32)
    out = pl.pallas_call(
        _otsu_kernel,
        grid_spec=pltpu.PrefetchScalarGridSpec(
            num_scalar_prefetch=1,
            grid=grid,
            in_specs=[
                pl.BlockSpec((bblk, 3, 1024), lambda i, s: (i, 0, 0)),
                pl.BlockSpec((bblk, 3, 1024), lambda i, s: (i, 0, 0)),
            ],
            out_specs=pl.BlockSpec((bblk, 1024), lambda i, s: (i, 0)),
        ),
        out_shape=jax.ShapeDtypeStruct((b, 1024), jnp.int32),
        compiler_params=pltpu.CompilerParams(
            dimension_semantics=("parallel",)),
    )(thresholds, xt, pt)
    return out.reshape(b, 32, 32)


def kernel(x, eps_noise, ew1, eb1, es1, eo1, ew2, eb2, es2, eo2,
           ew3, eb3, es3, eo3, wzm, bzm, wzl, bzl, dw, db,
           tw1, tb1, ds1, do1, tw2, tb2, ds2, do2,
           tw3, tb3, ds3, do3, tw4, tb4):
    relu = jax.nn.relu
    b = x.shape[0]
    h = relu(_conv2(x, ew1, 2) + eb1) * es1 + eo1
    h = relu(_conv2(h, ew2, 2) + eb2) * es2 + eo2
    h = relu(_conv2(h, ew3, 2) + eb3) * es3 + eo3
    h = h.reshape(b, -1)
    dvec = _dense_middle(h, eps_noise, wzm, bzm, wzl, bzl, dw, db, 256)
    d = dvec.reshape(b, 8, 8, 128)
    d = relu(_convT2(d, tw1, 1) + tb1) * ds1 + do1
    d = relu(_convT2(d, tw2, 2) + tb2) * ds2 + do2
    d = relu(_convT2(d, tw3, 2) + tb3) * ds3 + do3
    pre = _convT2(d, tw4, 1) + tb4
    return _otsu_mask(x, pre, 256)


# speedup vs baseline: 4.1373x; 4.1373x over previous
import jax
import jax.numpy as jnp
import numpy as np
from jax.experimental import pallas as pl
from jax.experimental.pallas import tpu as pltpu

_THRESHOLDS = [float(t) for t in np.arange(0.1, 0.91, 0.01).astype(np.float32)]
_HW = 32 * 32


def _conv2(a, w, s):
    return jax.lax.conv_general_dilated(
        a, w, (s, s), 'SAME', dimension_numbers=('NHWC', 'HWIO', 'NHWC'))


def _convT2(a, w, s):
    return jax.lax.conv_transpose(
        a, w, (s, s), 'SAME', dimension_numbers=('NHWC', 'HWIO', 'NHWC'))


def _mid_kernel(h_ref, eps_ref, wzm_ref, bzm_ref, wzl_ref, bzl_ref,
                dw_ref, db_ref, o_ref):
    h = h_ref[...]
    zm = jnp.dot(h, wzm_ref[...], preferred_element_type=jnp.float32)
    zm = jnp.maximum(zm + bzm_ref[...], 0.0)
    zl = jnp.dot(h, wzl_ref[...], preferred_element_type=jnp.float32)
    zl = jnp.maximum(zl + bzl_ref[...], 0.0)
    z = zm + jnp.exp(0.5 * zl) * eps_ref[...]
    d = jnp.dot(z, dw_ref[...], preferred_element_type=jnp.float32)
    o_ref[...] = jnp.maximum(d + db_ref[...], 0.0)


def _otsu_kernel(t_ref, x_ref, p_ref, o_ref):
    recon = jax.nn.sigmoid(p_ref[...])
    diff = x_ref[...] - recon
    err = jnp.mean(diff * diff, axis=0)

    zero = jnp.zeros((1, err.shape[1]), jnp.float32)

    def body(i, carry):
        sig_max, opti = carry
        t = t_ref[i]
        below = err < t
        e0 = jnp.where(below, err, 0.0)
        e1 = jnp.where(below, 0.0, err)
        n0 = jnp.sum((e0 != 0.0).astype(jnp.float32), axis=0, keepdims=True)
        n1 = jnp.sum((e1 != 0.0).astype(jnp.float32), axis=0, keepdims=True)
        s0 = jnp.sum(e0, axis=0, keepdims=True)
        s1 = jnp.sum(e1, axis=0, keepdims=True)
        m0 = (s0 / _HW) * ((n1 + n0) / n0)
        m0 = jnp.where(jnp.isnan(m0), 0.0, m0)
        m1 = (s1 / _HW) * ((n1 + n0) / n1)
        m1 = jnp.where(jnp.isnan(m1), 0.0, m1)
        p0 = n0 / (n0 + n1)
        p1 = n1 / (n0 + n1)
        sig_b = p0 * p1 * (m0 - m1) ** 2
        upd = sig_b >= sig_max
        return (jnp.where(upd, sig_b, sig_max), jnp.where(upd, t, opti))

    _, opti = jax.lax.fori_loop(0, len(_THRESHOLDS), body, (zero, zero))
    o_ref[...] = (err < opti).astype(jnp.int32)


def _dense_middle(h, eps, wzm, bzm, wzl, bzl, dw, db, bblk):
    b = h.shape[0]
    grid = (b // bblk,)
    return pl.pallas_call(
        _mid_kernel,
        grid=grid,
        in_specs=[
            pl.BlockSpec((bblk, 2048), lambda i: (i, 0)),
            pl.BlockSpec((bblk, 128), lambda i: (i, 0)),
            pl.BlockSpec((2048, 128), lambda i: (0, 0)),
            pl.BlockSpec((1, 128), lambda i: (0, 0)),
            pl.BlockSpec((2048, 128), lambda i: (0, 0)),
            pl.BlockSpec((1, 128), lambda i: (0, 0)),
            pl.BlockSpec((128, 8192), lambda i: (0, 0)),
            pl.BlockSpec((1, 8192), lambda i: (0, 0)),
        ],
        out_specs=pl.BlockSpec((bblk, 8192), lambda i: (i, 0)),
        out_shape=jax.ShapeDtypeStruct((b, 8192), jnp.float32),
        compiler_params=pltpu.CompilerParams(
            dimension_semantics=("parallel",)),
    )(h, eps, wzm, bzm.reshape(1, -1), wzl, bzl.reshape(1, -1),
      dw, db.reshape(1, -1))


def _otsu_mask(x, pre, bblk):
    b = x.shape[0]
    xt = x.reshape(b, 1024, 3).transpose(2, 1, 0)
    pt = pre.reshape(b, 1024, 3).transpose(2, 1, 0)
    grid = (b // bblk,)
    thresholds = jnp.asarray(np.arange(0.1, 0.91, 0.01), dtype=jnp.float32)
    out = pl.pallas_call(
        _otsu_kernel,
        grid_spec=pltpu.PrefetchScalarGridSpec(
            num_scalar_prefetch=1,
            grid=grid,
            in_specs=[
                pl.BlockSpec((3, 1024, bblk), lambda i, s: (0, 0, i)),
                pl.BlockSpec((3, 1024, bblk), lambda i, s: (0, 0, i)),
            ],
            out_specs=pl.BlockSpec((1024, bblk), lambda i, s: (0, i)),
        ),
        out_shape=jax.ShapeDtypeStruct((1024, b), jnp.int32),
        compiler_params=pltpu.CompilerParams(
            dimension_semantics=("parallel",)),
    )(thresholds, xt, pt)
    return out.T.reshape(b, 32, 32)


def kernel(x, eps_noise, ew1, eb1, es1, eo1, ew2, eb2, es2, eo2,
           ew3, eb3, es3, eo3, wzm, bzm, wzl, bzl, dw, db,
           tw1, tb1, ds1, do1, tw2, tb2, ds2, do2,
           tw3, tb3, ds3, do3, tw4, tb4):
    relu = jax.nn.relu
    b = x.shape[0]
    h = relu(_conv2(x, ew1, 2) + eb1) * es1 + eo1
    h = relu(_conv2(h, ew2, 2) + eb2) * es2 + eo2
    h = relu(_conv2(h, ew3, 2) + eb3) * es3 + eo3
    h = h.reshape(b, -1)
    dvec = _dense_middle(h, eps_noise, wzm, bzm, wzl, bzl, dw, db, 256)
    d = dvec.reshape(b, 8, 8, 128)
    d = relu(_convT2(d, tw1, 1) + tb1) * ds1 + do1
    d = relu(_convT2(d, tw2, 2) + tb2) * ds2 + do2
    d = relu(_convT2(d, tw3, 2) + tb3) * ds3 + do3
    pre = _convT2(d, tw4, 1) + tb4
    return _otsu_mask(x, pre, 256)
